# baseline (speedup 1.0000x reference)
import functools

import jax
import jax.numpy as jnp
from jax import lax
from jax.experimental import pallas as pl
from jax.experimental.pallas import tpu as pltpu
from jax.experimental.pallas import tpu_sc as plsc

D_MODEL_K = 64
SCALE_K = 8.0

_NC = 2
_NS = 16
_NW = _NC * _NS
_LANES = 16

_B_TOTAL = 4096 * 200
_B_PER_W = _B_TOTAL // _NW
_CHUNK = 1024
_N_CHUNKS = _B_PER_W // _CHUNK


def _emb_kernel(table_hbm, idx_hbm, out_hbm, idx_v, rows_v, sem):
    wid = lax.axis_index("s") * _NC + lax.axis_index("c")
    wbase = wid * _B_PER_W

    def chunk_body(g, carry):
        base = wbase + g * _CHUNK
        pltpu.sync_copy(idx_hbm.at[pl.ds(base, _CHUNK)], idx_v)
        pltpu.async_copy(table_hbm.at[idx_v], rows_v, sem).wait()

        def scale_row(i, c):
            for j in range(D_MODEL_K // _LANES):
                sl = pl.ds(j * _LANES, _LANES)
                rows_v[i, sl] = rows_v[i, sl] * SCALE_K
            return c

        lax.fori_loop(0, _CHUNK, scale_row, 0, unroll=4)
        pltpu.sync_copy(rows_v, out_hbm.at[pl.ds(base, _CHUNK)])
        return carry

    lax.fori_loop(0, _N_CHUNKS, chunk_body, 0)


@jax.jit
def _emb_call(idx_flat, table):
    mesh = plsc.VectorSubcoreMesh(core_axis_name="c", subcore_axis_name="s")
    run = functools.partial(
        pl.kernel,
        mesh=mesh,
        out_type=jax.ShapeDtypeStruct((_B_TOTAL, D_MODEL_K), jnp.float32),
        compiler_params=pltpu.CompilerParams(use_tc_tiling_on_sc=False),
        scratch_types=[
            pltpu.VMEM((_CHUNK,), jnp.int32),
            pltpu.VMEM((_CHUNK, D_MODEL_K), jnp.float32),
            pltpu.SemaphoreType.DMA,
        ],
    )(_emb_kernel)
    return run(table, idx_flat)


def kernel(x, table):
    idx_flat = x.reshape(-1).astype(jnp.int32)
    out = _emb_call(idx_flat, table)
    return out.reshape(x.shape + (D_MODEL_K,))


# speedup vs baseline: 1.0627x; 1.0627x over previous
import functools

import jax
import jax.numpy as jnp
from jax import lax
from jax.experimental import pallas as pl
from jax.experimental.pallas import tpu as pltpu
from jax.experimental.pallas import tpu_sc as plsc

D_MODEL_K = 64
SCALE_K = 8.0

_NC = 2
_NS = 16
_NW = _NC * _NS
_LANES = 16

_B_TOTAL = 4096 * 200
_B_PER_W = _B_TOTAL // _NW
_CHUNK = 400
_N_CHUNKS = _B_PER_W // _CHUNK
_NBUF = 4


def _emb_kernel(table_hbm, idx_hbm, out_hbm, idx_v,
                rb0, rb1, rb2, rb3, sg0, sg1, sg2, sg3,
                sw0, sw1, sw2, sw3):
    bufs = (rb0, rb1, rb2, rb3)
    sgs = (sg0, sg1, sg2, sg3)
    sws = (sw0, sw1, sw2, sw3)
    wid = lax.axis_index("s") * _NC + lax.axis_index("c")
    wbase = wid * _B_PER_W
    pltpu.sync_copy(idx_hbm.at[pl.ds(wbase, _B_PER_W)], idx_v)

    def start_gather(g, b):
        pltpu.async_copy(
            table_hbm.at[idx_v.at[pl.ds(g * _CHUNK, _CHUNK)]], bufs[b], sgs[b])

    def wait_gather(g, b):
        pltpu.make_async_copy(
            table_hbm.at[idx_v.at[pl.ds(g * _CHUNK, _CHUNK)]], bufs[b],
            sgs[b]).wait()

    def start_wb(g, b):
        pltpu.async_copy(
            bufs[b], out_hbm.at[pl.ds(wbase + g * _CHUNK, _CHUNK)], sws[b])

    def wait_wb(g, b):
        pltpu.make_async_copy(
            bufs[b], out_hbm.at[pl.ds(wbase + g * _CHUNK, _CHUNK)],
            sws[b]).wait()

    start_gather(0, 0)
    start_gather(1, 1)

    def quad(q, carry):
        for b in range(_NBUF):
            g = q * _NBUF + b
            bn = (b + 2) % _NBUF

            @pl.when(g >= 2)
            def _():
                wait_wb(g - 2, bn)

            @pl.when(g + 2 < _N_CHUNKS)
            def _():
                start_gather(g + 2, bn)

            wait_gather(g, b)

            @plsc.parallel_loop(0, _CHUNK, unroll=4)
            def _(i):
                for j in range(D_MODEL_K // _LANES):
                    sl = pl.ds(j * _LANES, _LANES)
                    bufs[b][i, sl] = bufs[b][i, sl] * SCALE_K

            start_wb(g, b)
        return carry

    lax.fori_loop(0, _N_CHUNKS // _NBUF, quad, 0)
    wait_wb(_N_CHUNKS - 2, (_N_CHUNKS - 2) % _NBUF)
    wait_wb(_N_CHUNKS - 1, (_N_CHUNKS - 1) % _NBUF)


@jax.jit
def _emb_call(idx_flat, table):
    mesh = plsc.VectorSubcoreMesh(core_axis_name="c", subcore_axis_name="s")
    run = functools.partial(
        pl.kernel,
        mesh=mesh,
        out_type=jax.ShapeDtypeStruct((_B_TOTAL, D_MODEL_K), jnp.float32),
        compiler_params=pltpu.CompilerParams(use_tc_tiling_on_sc=False),
        scratch_types=(
            [pltpu.VMEM((_B_PER_W,), jnp.int32)]
            + [pltpu.VMEM((_CHUNK, D_MODEL_K), jnp.float32)] * _NBUF
            + [pltpu.SemaphoreType.DMA] * (2 * _NBUF)
        ),
    )(_emb_kernel)
    return run(table, idx_flat)


def kernel(x, table):
    idx_flat = x.reshape(-1).astype(jnp.int32)
    out = _emb_call(idx_flat, table)
    return out.reshape(x.shape + (D_MODEL_K,))
